# baseline (speedup 1.0000x reference)
import functools

import jax
import jax.numpy as jnp
from jax import lax
from jax.experimental import pallas as pl
from jax.experimental.pallas import tpu as pltpu
from jax.experimental.pallas import tpu_sc as plsc

B = 4096
S = 200
D = 64
NUM_LABELS = 100

NC = 2
NS = 16
NW = NC * NS
CHUNK = 100
CPR = S // CHUNK
ROWS_PER_W = B // NW
CHUNKS_PER_W = ROWS_PER_W * CPR
LANES = 16
DV = D // LANES

_mesh = plsc.VectorSubcoreMesh(core_axis_name="c", subcore_axis_name="s")


@functools.partial(
    pl.kernel,
    out_type=jax.ShapeDtypeStruct((B, D), jnp.float32),
    mesh=_mesh,
    scratch_types=[
        pltpu.VMEM((CHUNKS_PER_W, CHUNK), jnp.int32),
        pltpu.VMEM((CHUNK, D), jnp.float32),
        pltpu.VMEM((CHUNK, D), jnp.float32),
        pltpu.VMEM((ROWS_PER_W, D), jnp.float32),
        pltpu.SemaphoreType.DMA,
        pltpu.SemaphoreType.DMA,
    ],
    compiler_params=pltpu.CompilerParams(use_tc_tiling_on_sc=False),
)
def _pool_sc(tok_hbm, emb_hbm, out_hbm, idx_v, buf0, buf1, acc_v, sem0, sem1):
    wid = lax.axis_index("s") * NC + lax.axis_index("c")
    bufs = (buf0, buf1)
    sems = (sem0, sem1)

    pltpu.sync_copy(tok_hbm.at[pl.ds(wid * CHUNKS_PER_W, CHUNKS_PER_W)], idx_v)

    for p in range(2):
        pltpu.async_copy(emb_hbm.at[idx_v.at[p]], bufs[p], sems[p])

    def pair_body(k, _):
        acc = [jnp.zeros((LANES,), jnp.float32) for _ in range(DV)]
        for p in range(2):
            c = 2 * k + p
            pltpu.make_async_copy(emb_hbm.at[idx_v.at[c]], bufs[p], sems[p]).wait()
            for t in range(CHUNK):
                for j in range(DV):
                    acc[j] = acc[j] + bufs[p][t, pl.ds(j * LANES, LANES)]
            @pl.when(c + 2 < CHUNKS_PER_W)
            def _():
                pltpu.async_copy(emb_hbm.at[idx_v.at[c + 2]], bufs[p], sems[p])
        for j in range(DV):
            acc_v[k, pl.ds(j * LANES, LANES)] = acc[j]
        return 0

    lax.fori_loop(0, ROWS_PER_W, pair_body, 0)

    pltpu.sync_copy(acc_v, out_hbm.at[pl.ds(wid * ROWS_PER_W, ROWS_PER_W)])


def _cls_tc(pooled_ref, w_ref, b_ref, logits_ref, preds_ref):
    pooled = pooled_ref[...] * (1.0 / S)
    logits = (
        jnp.dot(pooled, w_ref[...], preferred_element_type=jnp.float32)
        + b_ref[...]
    )
    logits_ref[...] = logits
    mx = jnp.max(logits, axis=1, keepdims=True)
    lbl = lax.broadcasted_iota(jnp.int32, logits.shape, 1)
    cand = jnp.where(logits == mx, lbl, NUM_LABELS)
    preds_ref[...] = jnp.min(cand, axis=1, keepdims=True)


_cls_call = pl.pallas_call(
    _cls_tc,
    out_shape=(
        jax.ShapeDtypeStruct((B, NUM_LABELS), jnp.float32),
        jax.ShapeDtypeStruct((B, 1), jnp.int32),
    ),
)


@jax.jit
def kernel(token_ids, emb_table, cls_w, cls_b):
    tok = token_ids.reshape(NW * CHUNKS_PER_W, CHUNK).astype(jnp.int32)
    pooled_sum = _pool_sc(tok, emb_table)
    logits, preds = _cls_call(pooled_sum, cls_w, cls_b.reshape(1, NUM_LABELS))
    return logits, preds.reshape(B)
